# baseline (speedup 1.0000x reference)
import jax
import jax.numpy as jnp
from jax.experimental import pallas as pl

VOCAB = 100000
MAX_LENGTH = 15
TOPK = 40
SOS_IDX = 1
EOS_IDX = 2

BLKR = 8192
NBLK = 13
GRP = 32
GPB = BLKR // GRP
NGRP = NBLK * GPB
KGRP = 42


def _vm_body(xt_ref, m_ref, l_ref, vm_ref):
    j = pl.program_id(0)
    xt = xt_ref[...]
    m = m_ref[0:1, :]
    L = l_ref[0:1, :]
    p = jnp.exp((xt - m) - L)
    row = jax.lax.broadcasted_iota(jnp.int32, (BLKR, 128), 0)
    valid = (j * BLKR + row) < VOCAB
    p = jnp.where(valid, p, 0.0)
    rows = [jnp.max(p[GRP * g:GRP * (g + 1), :], axis=0, keepdims=True)
            for g in range(GPB)]
    vm_ref[...] = jnp.concatenate(rows, axis=0)


def _group_max_T(logits_T, m_row, L_row):
    return pl.pallas_call(
        _vm_body,
        grid=(NBLK,),
        in_specs=[
            pl.BlockSpec((BLKR, 128), lambda j: (j, 0)),
            pl.BlockSpec((8, 128), lambda j: (0, 0)),
            pl.BlockSpec((8, 128), lambda j: (0, 0)),
        ],
        out_specs=pl.BlockSpec((GPB, 128), lambda j: (j, 0)),
        out_shape=jax.ShapeDtypeStruct((NGRP, 128), jnp.float32),
    )(logits_T, m_row, L_row)


def _gru(x, h, W_ih, W_hh, b_ih, b_hh):
    gi = x @ W_ih + b_ih
    gh = h @ W_hh + b_hh
    i_r, i_z, i_n = jnp.split(gi, 3, axis=-1)
    h_r, h_z, h_n = jnp.split(gh, 3, axis=-1)
    r = jax.nn.sigmoid(i_r + h_r)
    z = jax.nn.sigmoid(i_z + h_z)
    n = jnp.tanh(i_n + r * h_n)
    return (1.0 - z) * n + z * h


def _topk_via_groups(logits, m, L):
    B = logits.shape[0]
    logits_T = logits.T
    m8 = jnp.broadcast_to(m.T, (8, B))
    L8 = jnp.broadcast_to(L.T, (8, B))
    vm_T = _group_max_T(logits_T, m8, L8)
    vm = vm_T.T
    _, gids = jax.lax.top_k(vm, KGRP)
    gids = jnp.sort(gids, axis=1)
    cand_idx = (gids[:, :, None] * GRP
                + jnp.arange(GRP, dtype=jnp.int32)[None, None, :]).reshape(B, -1)
    cand_idx_c = jnp.minimum(cand_idx, VOCAB - 1)
    x_cand = jnp.take_along_axis(logits_T, cand_idx_c.T, axis=0).T
    p_cand = jnp.exp((x_cand - m) - L)
    p_cand = jnp.where(cand_idx < VOCAB, p_cand, 0.0)
    tk_p, tk_pos = jax.lax.top_k(p_cand, TOPK)
    tk_i = jnp.take_along_axis(cand_idx, tk_pos, axis=1)
    p40 = tk_p[:, TOPK - 1:TOPK]
    ok = jnp.all(jnp.sum((vm >= p40).astype(jnp.int32), axis=1) <= KGRP)

    def _fallback(_):
        wp = jnp.exp((logits - m) - L)
        fp, fi = jax.lax.top_k(wp, TOPK)
        return (fp, fi)

    def _fast(_):
        return (tk_p, tk_i)

    return jax.lax.cond(ok, _fast, _fallback, operand=None)


def kernel(users, items, user_emb, item_emb, tok_emb,
           W_init, b_init, W_ih, W_hh, b_ih, b_hh, W_out, b_out):
    B = users.shape[0]
    ui = jnp.concatenate([user_emb[users], item_emb[items]], axis=-1)
    hidden = jnp.tanh(ui @ W_init + b_init)
    word_var = jnp.full((B,), SOS_IDX, dtype=jnp.int32)
    rvw_lens = jnp.zeros((B,), dtype=jnp.int32)
    words = []
    probs = []
    base_key = jax.random.key(42)
    for i in range(MAX_LENGTH):
        x = tok_emb[word_var]
        hidden = _gru(x, hidden, W_ih, W_hh, b_ih, b_hh)
        logits = hidden @ W_out + b_out
        m = jnp.max(logits, axis=-1, keepdims=True)
        shifted = logits - m
        L = jnp.log(jnp.sum(jnp.exp(shifted), axis=-1, keepdims=True))
        tk_p, tk_i = _topk_via_groups(logits, m, L)
        samp = jax.random.categorical(jax.random.fold_in(base_key, i),
                                      jnp.log(tk_p + 1e-20), axis=-1)
        prob_var = jnp.take_along_axis(tk_p, samp[:, None], axis=-1)[:, 0]
        word_var = jnp.take_along_axis(tk_i, samp[:, None], axis=-1)[:, 0]
        words.append(word_var)
        probs.append(prob_var)
        is_eos = word_var == EOS_IDX
        not_end = rvw_lens == 0
        if i != MAX_LENGTH - 1:
            rvw_lens = jnp.where(not_end & is_eos, i + 1, rvw_lens)
        else:
            rvw_lens = jnp.where(not_end, MAX_LENGTH, rvw_lens)
    return jnp.stack(words, axis=0), jnp.stack(probs, axis=0), rvw_lens


# speedup vs baseline: 2.7051x; 2.7051x over previous
import jax
import jax.numpy as jnp
from jax.experimental import pallas as pl

VOCAB = 100000
MAX_LENGTH = 15
TOPK = 40
SOS_IDX = 1
EOS_IDX = 2

BLKR = 8192
NBLK = 13
GRP = 32
GPB = BLKR // GRP
NGRP = NBLK * GPB
KGRP = 44


def _vm_body(xt_ref, m_ref, l_ref, vm_ref):
    j = pl.program_id(0)
    xt = xt_ref[...]
    m = m_ref[0:1, :]
    L = l_ref[0:1, :]
    p = jnp.exp((xt - m) - L)
    row = jax.lax.broadcasted_iota(jnp.int32, (BLKR, 128), 0)
    valid = (j * BLKR + row) < VOCAB
    p = jnp.where(valid, p, 0.0)
    rows = [jnp.max(p[GRP * g:GRP * (g + 1), :], axis=0, keepdims=True)
            for g in range(GPB)]
    vm_ref[...] = jnp.concatenate(rows, axis=0)


def _group_max_T(logits_T, m_row, L_row):
    return pl.pallas_call(
        _vm_body,
        grid=(NBLK,),
        in_specs=[
            pl.BlockSpec((BLKR, 128), lambda j: (j, 0)),
            pl.BlockSpec((8, 128), lambda j: (0, 0)),
            pl.BlockSpec((8, 128), lambda j: (0, 0)),
        ],
        out_specs=pl.BlockSpec((GPB, 128), lambda j: (j, 0)),
        out_shape=jax.ShapeDtypeStruct((NGRP, 128), jnp.float32),
    )(logits_T, m_row, L_row)


def _gru(x, h, W_ih, W_hh, b_ih, b_hh):
    gi = x @ W_ih + b_ih
    gh = h @ W_hh + b_hh
    i_r, i_z, i_n = jnp.split(gi, 3, axis=-1)
    h_r, h_z, h_n = jnp.split(gh, 3, axis=-1)
    r = jax.nn.sigmoid(i_r + h_r)
    z = jax.nn.sigmoid(i_z + h_z)
    n = jnp.tanh(i_n + r * h_n)
    return (1.0 - z) * n + z * h


def _topk_via_groups(logits, m, L):
    B = logits.shape[0]
    logits_T = logits.T
    m8 = jnp.broadcast_to(m.T, (8, B))
    L8 = jnp.broadcast_to(L.T, (8, B))
    vm_T = _group_max_T(logits_T, m8, L8)
    vm = vm_T.T
    _, gids = jax.lax.top_k(vm, KGRP)
    gids = jnp.sort(gids, axis=1)
    cand_idx = (gids[:, :, None] * GRP
                + jnp.arange(GRP, dtype=jnp.int32)[None, None, :]).reshape(B, -1)
    cand_idx_c = jnp.minimum(cand_idx, VOCAB - 1)
    x_cand = jnp.take_along_axis(logits_T, cand_idx_c.T, axis=0).T
    p_cand = jnp.exp((x_cand - m) - L)
    p_cand = jnp.where(cand_idx < VOCAB, p_cand, 0.0)
    tk_p, tk_pos = jax.lax.top_k(p_cand, TOPK)
    tk_i = jnp.take_along_axis(cand_idx, tk_pos, axis=1)
    p40 = tk_p[:, TOPK - 1:TOPK]
    ok = jnp.all(jnp.sum((vm >= p40).astype(jnp.int32), axis=1) <= KGRP)

    def _fallback(_):
        wp = jnp.exp((logits - m) - L)
        fp, fi = jax.lax.top_k(wp, TOPK)
        return (fp, fi)

    def _fast(_):
        return (tk_p, tk_i)

    return jax.lax.cond(ok, _fast, _fallback, operand=None)


def kernel(users, items, user_emb, item_emb, tok_emb,
           W_init, b_init, W_ih, W_hh, b_ih, b_hh, W_out, b_out):
    B = users.shape[0]
    ui = jnp.concatenate([user_emb[users], item_emb[items]], axis=-1)
    hidden = jnp.tanh(ui @ W_init + b_init)
    word_var = jnp.full((B,), SOS_IDX, dtype=jnp.int32)
    rvw_lens = jnp.zeros((B,), dtype=jnp.int32)
    words = []
    probs = []
    base_key = jax.random.key(42)
    for i in range(MAX_LENGTH):
        x = tok_emb[word_var]
        hidden = _gru(x, hidden, W_ih, W_hh, b_ih, b_hh)
        logits = hidden @ W_out + b_out
        m = jnp.max(logits, axis=-1, keepdims=True)
        shifted = logits - m
        L = jnp.log(jnp.sum(jnp.exp(shifted), axis=-1, keepdims=True))
        tk_p, tk_i = _topk_via_groups(logits, m, L)
        samp = jax.random.categorical(jax.random.fold_in(base_key, i),
                                      jnp.log(tk_p + 1e-20), axis=-1)
        prob_var = jnp.take_along_axis(tk_p, samp[:, None], axis=-1)[:, 0]
        word_var = jnp.take_along_axis(tk_i, samp[:, None], axis=-1)[:, 0]
        words.append(word_var)
        probs.append(prob_var)
        is_eos = word_var == EOS_IDX
        not_end = rvw_lens == 0
        if i != MAX_LENGTH - 1:
            rvw_lens = jnp.where(not_end & is_eos, i + 1, rvw_lens)
        else:
            rvw_lens = jnp.where(not_end, MAX_LENGTH, rvw_lens)
    return jnp.stack(words, axis=0), jnp.stack(probs, axis=0), rvw_lens
